# Initial kernel scaffold; baseline (speedup 1.0000x reference)
import jax
import jax.numpy as jnp
from jax.experimental import pallas as pl


def kernel(input_ids, embed_table):
    raise NotImplementedError("write your pallas kernel here")


# speedup vs baseline: 1.6270x; 1.6270x over previous
import functools

import jax
import jax.numpy as jnp
from jax import lax
from jax.experimental import pallas as pl
from jax.experimental.pallas import tpu as pltpu
from jax.experimental.pallas import tpu_sc as plsc

_NUM_CORES = 2
_NUM_SUBCORES = 16
_NUM_WORKERS = _NUM_CORES * _NUM_SUBCORES


@functools.partial(jax.jit, static_argnames=("n", "d"))
def _sc_gather(ids_flat, table, *, n, d):
    bpw = n // _NUM_WORKERS
    chunk = 32
    nchunk = bpw // chunk
    mesh = plsc.VectorSubcoreMesh(core_axis_name="c", subcore_axis_name="s")

    @functools.partial(
        pl.kernel,
        mesh=mesh,
        out_type=jax.ShapeDtypeStruct((n, d), table.dtype),
        scratch_types=[
            pltpu.VMEM((bpw,), jnp.int32),
            pltpu.VMEM((chunk, d), table.dtype),
            pltpu.SemaphoreType.DMA,
            pltpu.SemaphoreType.DMA,
        ],
    )
    def run(ids_hbm, table_hbm, out_hbm, idx_v, buf, sem_in, sem_out):
        wid = lax.axis_index("s") * _NUM_CORES + lax.axis_index("c")
        base = wid * bpw
        pltpu.sync_copy(ids_hbm.at[pl.ds(base, bpw)], idx_v)

        def step(g, _):
            off = g * chunk
            pltpu.async_copy(
                table_hbm.at[idx_v.at[pl.ds(off, chunk)]], buf, sem_in
            ).wait()
            pltpu.async_copy(
                buf, out_hbm.at[pl.ds(base + off, chunk)], sem_out
            ).wait()
            return ()

        lax.fori_loop(0, nchunk, step, ())

    return run(ids_flat, table)


def kernel(input_ids, embed_table):
    n = input_ids.size
    d = embed_table.shape[1]
    ids_flat = input_ids.reshape(-1).astype(jnp.int32)
    out = _sc_gather(ids_flat, embed_table, n=n, d=d)
    return out.reshape(input_ids.shape + (d,))


# speedup vs baseline: 1.7718x; 1.0890x over previous
import functools

import jax
import jax.numpy as jnp
from jax import lax
from jax.experimental import pallas as pl
from jax.experimental.pallas import tpu as pltpu
from jax.experimental.pallas import tpu_sc as plsc

_NUM_CORES = 2
_NUM_SUBCORES = 16
_NUM_WORKERS = _NUM_CORES * _NUM_SUBCORES


@functools.partial(jax.jit, static_argnames=("n", "d"))
def _sc_gather(ids_flat, table, *, n, d):
    bpw = n // _NUM_WORKERS
    chunk = 16
    nchunk = bpw // chunk
    mesh = plsc.VectorSubcoreMesh(core_axis_name="c", subcore_axis_name="s")

    @functools.partial(
        pl.kernel,
        mesh=mesh,
        out_type=jax.ShapeDtypeStruct((n, d), table.dtype),
        scratch_types=[
            pltpu.VMEM((bpw,), jnp.int32),
            pltpu.VMEM((chunk, d), table.dtype),
            pltpu.VMEM((chunk, d), table.dtype),
            pltpu.SemaphoreType.DMA,
            pltpu.SemaphoreType.DMA,
            pltpu.SemaphoreType.DMA,
            pltpu.SemaphoreType.DMA,
        ],
    )
    def run(ids_hbm, table_hbm, out_hbm, idx_v, buf0, buf1,
            sin0, sin1, sout0, sout1):
        wid = lax.axis_index("s") * _NUM_CORES + lax.axis_index("c")
        base = wid * bpw
        pltpu.sync_copy(ids_hbm.at[pl.ds(base, bpw)], idx_v)

        def gather(g, buf, sem):
            return pltpu.make_async_copy(
                table_hbm.at[idx_v.at[pl.ds(g * chunk, chunk)]], buf, sem)

        def put(g, buf, sem):
            return pltpu.make_async_copy(
                buf, out_hbm.at[pl.ds(base + g * chunk, chunk)], sem)

        gather(0, buf0, sin0).start()
        gather(1, buf1, sin1).start()
        gather(0, buf0, sin0).wait()
        put(0, buf0, sout0).start()

        def step(s, _):
            g1 = 2 * s + 1
            g2 = 2 * s + 2
            put(g2 - 2, buf0, sout0).wait()
            gather(g2, buf0, sin0).start()
            gather(g1, buf1, sin1).wait()
            put(g1, buf1, sout1).start()
            put(g1, buf1, sout1).wait()
            gather(g2 + 1, buf1, sin1).start()
            gather(g2, buf0, sin0).wait()
            put(g2, buf0, sout0).start()
            return ()

        lax.fori_loop(0, nchunk // 2 - 1, step, ())

        g_last = nchunk - 1
        gather(g_last, buf1, sin1).wait()
        put(g_last, buf1, sout1).start()
        put(g_last - 1, buf0, sout0).wait()
        put(g_last, buf1, sout1).wait()

    return run(ids_flat, table)


def kernel(input_ids, embed_table):
    n = input_ids.size
    d = embed_table.shape[1]
    ids_flat = input_ids.reshape(-1).astype(jnp.int32)
    out = _sc_gather(ids_flat, embed_table, n=n, d=d)
    return out.reshape(input_ids.shape + (d,))
